# baseline (speedup 1.0000x reference)
import functools

import jax
import jax.numpy as jnp
import numpy as np
from jax.experimental import pallas as pl
from jax.experimental.pallas import tpu as pltpu

_ANCHORS = np.array([[10.0, 13.0], [16.0, 30.0], [33.0, 23.0]], dtype=np.float32)
_NUM_CLASSES = 80
_IMG_DIM = 608.0
_NA = 3


def _yolo_body(x_ref, o_ref, *, nG, stride):
    a = pl.program_id(1)
    attrs = x_ref.shape[1] // _NA
    v = x_ref[0, pl.ds(a * attrs, attrs)]

    gy = jax.lax.broadcasted_iota(jnp.int32, (1, nG, 1), 1).astype(jnp.float32)
    gx = jax.lax.broadcasted_iota(jnp.int32, (1, 1, nG), 2).astype(jnp.float32)

    sig = jax.nn.sigmoid(v)

    aw = jnp.where(a == 0, _ANCHORS[0, 0], jnp.where(a == 1, _ANCHORS[1, 0], _ANCHORS[2, 0]))
    ah = jnp.where(a == 0, _ANCHORS[0, 1], jnp.where(a == 1, _ANCHORS[1, 1], _ANCHORS[2, 1]))

    bx = (sig[0] + gx[0]) * stride
    by = (sig[1] + gy[0]) * stride
    bw = jnp.exp(v[2]) * aw
    bh = jnp.exp(v[3]) * ah
    val = jnp.concatenate([jnp.stack([bx, by, bw, bh], axis=0), sig[4:]], axis=0)

    S = nG * nG
    o_ref[0] = val.reshape(attrs, S).T


def kernel(x):
    nB, C, nG, _ = x.shape
    nA = _NA
    attrs = C // nA
    S = nG * nG
    stride = _IMG_DIM / nG

    return pl.pallas_call(
        functools.partial(_yolo_body, nG=nG, stride=stride),
        grid=(nB, nA),
        in_specs=[pl.BlockSpec((1, C, nG, nG), lambda b, a: (b, 0, 0, 0))],
        out_specs=pl.BlockSpec((1, S, attrs), lambda b, a: (b, a, 0)),
        out_shape=jax.ShapeDtypeStruct((nB, nA * S, attrs), jnp.float32),
        compiler_params=pltpu.CompilerParams(dimension_semantics=("parallel", "arbitrary")),
    )(x)


# speedup vs baseline: 1.1468x; 1.1468x over previous
import functools

import jax
import jax.numpy as jnp
import numpy as np
from jax.experimental import pallas as pl
from jax.experimental.pallas import tpu as pltpu

_ANCHORS = np.array([[10.0, 13.0], [16.0, 30.0], [33.0, 23.0]], dtype=np.float32)
_NUM_CLASSES = 80
_IMG_DIM = 608.0
_NA = 3


def _yolo_body(x_ref, o_ref, *, nG, stride):
    v = x_ref[0]
    nc = v.shape[0]
    attrs = nc // _NA

    rows = jax.lax.broadcasted_iota(jnp.int32, (nc, 1, 1), 0)
    r = rows % attrs
    gy = jax.lax.broadcasted_iota(jnp.int32, (1, nG, 1), 1).astype(jnp.float32)
    gx = jax.lax.broadcasted_iota(jnp.int32, (1, 1, nG), 2).astype(jnp.float32)

    sig = jax.nn.sigmoid(v)
    ex = jnp.exp(v)

    aw = jnp.where(rows < attrs, _ANCHORS[0, 0], jnp.where(rows < 2 * attrs, _ANCHORS[1, 0], _ANCHORS[2, 0]))
    ah = jnp.where(rows < attrs, _ANCHORS[0, 1], jnp.where(rows < 2 * attrs, _ANCHORS[1, 1], _ANCHORS[2, 1]))

    val = jnp.where(
        r == 0,
        (sig + gx) * stride,
        jnp.where(
            r == 1,
            (sig + gy) * stride,
            jnp.where(r == 2, ex * aw, jnp.where(r == 3, ex * ah, sig)),
        ),
    )
    S = nG * nG
    w = val.reshape(_NA, attrs, S)
    o_ref[0] = jnp.swapaxes(w, 1, 2).reshape(_NA * S, attrs)


def kernel(x):
    nB, C, nG, _ = x.shape
    nA = _NA
    attrs = C // nA
    S = nG * nG
    stride = _IMG_DIM / nG

    return pl.pallas_call(
        functools.partial(_yolo_body, nG=nG, stride=stride),
        grid=(nB,),
        in_specs=[pl.BlockSpec((1, C, nG, nG), lambda b: (b, 0, 0, 0))],
        out_specs=pl.BlockSpec((1, nA * S, attrs), lambda b: (b, 0, 0)),
        out_shape=jax.ShapeDtypeStruct((nB, nA * S, attrs), jnp.float32),
        compiler_params=pltpu.CompilerParams(dimension_semantics=("parallel",)),
    )(x)
